# baseline (speedup 1.0000x reference)
import jax
import jax.numpy as jnp
from jax.experimental import pallas as pl

_N, _H, _NH, _DH = 2048, 512, 8, 64
_BA = 256
_BQ = 256
_NEG = -1e9
_SCALE = 0.125


def _embed_qkv(px_ref, sx_ref, Ws_ref, bs_ref, g_ref, b_ref, Wp_ref,
               Wq_ref, Wk_ref, Wv_ref,
               h_ref, q_ref, k_ref, v_ref, fq_ref, gk_ref):
    px = px_ref[...]
    sx = sx_ref[...]
    h = jax.lax.dot_general(sx, Ws_ref[...], (((1,), (0,)), ((), ())),
                            preferred_element_type=jnp.float32)
    h = h + bs_ref[...]
    mu = jnp.mean(h, axis=1, keepdims=True)
    var = jnp.mean((h - mu) ** 2, axis=1, keepdims=True)
    h = (h - mu) / jnp.sqrt(var + 1e-5) * g_ref[...] + b_ref[...]
    h = h * jax.nn.sigmoid(h)
    h = h + jax.lax.dot_general(px, Wp_ref[...], (((1,), (0,)), ((), ())),
                                preferred_element_type=jnp.float32)
    h_ref[...] = h
    hb = h.astype(jnp.bfloat16)
    q_ref[...] = jax.lax.dot_general(hb, Wq_ref[...], (((1,), (0,)), ((), ())),
                                     preferred_element_type=jnp.float32
                                     ).astype(jnp.bfloat16)
    k_ref[...] = jax.lax.dot_general(hb, Wk_ref[...], (((1,), (0,)), ((), ())),
                                     preferred_element_type=jnp.float32
                                     ).astype(jnp.bfloat16)
    v_ref[...] = jax.lax.dot_general(hb, Wv_ref[...], (((1,), (0,)), ((), ())),
                                     preferred_element_type=jnp.float32
                                     ).astype(jnp.bfloat16)
    d = (px[:, 6:7] > 0).astype(jnp.int32)
    a = (px[:, 7:8] > 0).astype(jnp.int32)
    ar = (sx[:, 1:2] > 0).astype(jnp.int32)
    fq_ref[...] = d + 2 * a + 4 * ar
    gk_ref[...] = 2 * d + a + 4 * ar


def _attn(fq_ref, gk_ref, nc_ref, nr_ref, q_ref, k_ref, v_ref, o_ref):
    q = q_ref[...]
    k = k_ref[...]
    v = v_ref[...]
    mask = ((fq_ref[...] & gk_ref[...]) != 0) & (nc_ref[...] != nr_ref[...])
    for hh in range(_NH):
        sl = slice(hh * _DH, (hh + 1) * _DH)
        s = jax.lax.dot_general(q[:, sl], k[:, sl], (((1,), (1,)), ((), ())),
                                preferred_element_type=jnp.float32)
        s = jnp.where(mask, s * _SCALE, _NEG)
        m = jnp.max(s, axis=1, keepdims=True)
        p = jnp.exp(s - m)
        l = jnp.sum(p, axis=1, keepdims=True)
        ctx = jax.lax.dot_general(p.astype(jnp.bfloat16), v[:, sl],
                                  (((1,), (0,)), ((), ())),
                                  preferred_element_type=jnp.float32)
        o_ref[:, sl] = ctx / l


def _out_proj(h_ref, c_ref, Wo_ref, o_ref):
    o_ref[...] = h_ref[...] + jax.lax.dot_general(
        c_ref[...].astype(jnp.bfloat16), Wo_ref[...], (((1,), (0,)), ((), ())),
        preferred_element_type=jnp.float32)


def kernel(physics_x, structural_x, W_struct, b_struct, gamma, beta,
           W_phys, Wq, Wk, Wv, Wo, atom_to_nuc):
    nuc_col = atom_to_nuc.astype(jnp.int32).reshape(_N, 1)

    h, q, k, v, fq, gk = pl.pallas_call(
        _embed_qkv,
        grid=(_N // _BA,),
        in_specs=[
            pl.BlockSpec((_BA, 10), lambda i: (i, 0)),
            pl.BlockSpec((_BA, 4), lambda i: (i, 0)),
            pl.BlockSpec((4, _H), lambda i: (0, 0)),
            pl.BlockSpec((1, _H), lambda i: (0, 0)),
            pl.BlockSpec((1, _H), lambda i: (0, 0)),
            pl.BlockSpec((1, _H), lambda i: (0, 0)),
            pl.BlockSpec((10, _H), lambda i: (0, 0)),
            pl.BlockSpec((_H, _H), lambda i: (0, 0)),
            pl.BlockSpec((_H, _H), lambda i: (0, 0)),
            pl.BlockSpec((_H, _H), lambda i: (0, 0)),
        ],
        out_specs=[
            pl.BlockSpec((_BA, _H), lambda i: (i, 0)),
            pl.BlockSpec((_BA, _H), lambda i: (i, 0)),
            pl.BlockSpec((_BA, _H), lambda i: (i, 0)),
            pl.BlockSpec((_BA, _H), lambda i: (i, 0)),
            pl.BlockSpec((_BA, 1), lambda i: (i, 0)),
            pl.BlockSpec((_BA, 1), lambda i: (i, 0)),
        ],
        out_shape=[
            jax.ShapeDtypeStruct((_N, _H), jnp.float32),
            jax.ShapeDtypeStruct((_N, _H), jnp.bfloat16),
            jax.ShapeDtypeStruct((_N, _H), jnp.bfloat16),
            jax.ShapeDtypeStruct((_N, _H), jnp.bfloat16),
            jax.ShapeDtypeStruct((_N, 1), jnp.int32),
            jax.ShapeDtypeStruct((_N, 1), jnp.int32),
        ],
    )(physics_x, structural_x, W_struct, b_struct.reshape(1, _H),
      gamma.reshape(1, _H), beta.reshape(1, _H), W_phys,
      Wq.astype(jnp.bfloat16), Wk.astype(jnp.bfloat16),
      Wv.astype(jnp.bfloat16))

    gk_row = gk.reshape(1, _N)
    nuc_row = nuc_col.reshape(1, _N)

    ctx = pl.pallas_call(
        _attn,
        grid=(_N // _BQ,),
        in_specs=[
            pl.BlockSpec((_BQ, 1), lambda i: (i, 0)),
            pl.BlockSpec((1, _N), lambda i: (0, 0)),
            pl.BlockSpec((_BQ, 1), lambda i: (i, 0)),
            pl.BlockSpec((1, _N), lambda i: (0, 0)),
            pl.BlockSpec((_BQ, _H), lambda i: (i, 0)),
            pl.BlockSpec((_N, _H), lambda i: (0, 0)),
            pl.BlockSpec((_N, _H), lambda i: (0, 0)),
        ],
        out_specs=pl.BlockSpec((_BQ, _H), lambda i: (i, 0)),
        out_shape=jax.ShapeDtypeStruct((_N, _H), jnp.float32),
    )(fq, gk_row, nuc_col, nuc_row, q, k, v)

    out = pl.pallas_call(
        _out_proj,
        grid=(_N // _BA,),
        in_specs=[
            pl.BlockSpec((_BA, _H), lambda i: (i, 0)),
            pl.BlockSpec((_BA, _H), lambda i: (i, 0)),
            pl.BlockSpec((_H, _H), lambda i: (0, 0)),
        ],
        out_specs=pl.BlockSpec((_BA, _H), lambda i: (i, 0)),
        out_shape=jax.ShapeDtypeStruct((_N, _H), jnp.float32),
    )(h, ctx, Wo.astype(jnp.bfloat16))
    return out


# speedup vs baseline: 1.1136x; 1.1136x over previous
import jax
import jax.numpy as jnp
from jax.experimental import pallas as pl

_N, _H, _NH, _DH = 2048, 512, 8, 64
_BA = 256
_BQ = 256
_NEG = -1e9
_SCALE = 0.125


def _embed_qkv(px_ref, sx_ref, Ws_ref, bs_ref, g_ref, b_ref, Wp_ref,
               Wq_ref, Wk_ref, Wv_ref,
               h_ref, q_ref, k_ref, v_ref, fq_ref, gk_ref):
    px = px_ref[...]
    sx = sx_ref[...]
    h = jax.lax.dot_general(sx, Ws_ref[...], (((1,), (0,)), ((), ())),
                            preferred_element_type=jnp.float32)
    h = h + bs_ref[...]
    mu = jnp.mean(h, axis=1, keepdims=True)
    var = jnp.mean((h - mu) ** 2, axis=1, keepdims=True)
    h = (h - mu) / jnp.sqrt(var + 1e-5) * g_ref[...] + b_ref[...]
    h = h * jax.nn.sigmoid(h)
    h = h + jax.lax.dot_general(px, Wp_ref[...], (((1,), (0,)), ((), ())),
                                preferred_element_type=jnp.float32)
    h_ref[...] = h
    hb = h.astype(jnp.bfloat16)
    q_ref[...] = (jax.lax.dot_general(hb, Wq_ref[...], (((1,), (0,)), ((), ())),
                                      preferred_element_type=jnp.float32)
                  * _SCALE).astype(jnp.bfloat16)
    k_ref[...] = jax.lax.dot_general(hb, Wk_ref[...], (((1,), (0,)), ((), ())),
                                     preferred_element_type=jnp.float32
                                     ).astype(jnp.bfloat16)
    v_ref[...] = jax.lax.dot_general(hb, Wv_ref[...], (((1,), (0,)), ((), ())),
                                     preferred_element_type=jnp.float32)
    d = (px[:, 6:7] > 0).astype(jnp.int32)
    a = (px[:, 7:8] > 0).astype(jnp.int32)
    ar = (sx[:, 1:2] > 0).astype(jnp.int32)
    fq_ref[...] = d + 2 * a + 4 * ar
    gk_ref[...] = 2 * d + a + 4 * ar


def _attn(fq_ref, gk_ref, nc_ref, nr_ref, q_ref, k_ref, v_ref, h_ref, Wo_ref,
          o_ref):
    q = q_ref[...]
    k = k_ref[...]
    v = v_ref[...]
    mask = ((fq_ref[...] & gk_ref[...]) != 0) & (nc_ref[...] != nr_ref[...])
    parts = []
    for hh in range(_NH):
        sl = slice(hh * _DH, (hh + 1) * _DH)
        s = jax.lax.dot_general(q[:, sl], k[:, sl], (((1,), (1,)), ((), ())),
                                preferred_element_type=jnp.float32)
        s = jnp.where(mask, s, _NEG)
        m = jnp.max(s, axis=1, keepdims=True)
        p = jnp.exp(s - m)
        l = jnp.sum(p, axis=1, keepdims=True)
        ctx = jax.lax.dot_general(p, v[:, sl], (((1,), (0,)), ((), ())),
                                  preferred_element_type=jnp.float32)
        parts.append(ctx / l)
    ctx_all = jnp.concatenate(parts, axis=1).astype(jnp.bfloat16)
    o_ref[...] = h_ref[...] + jax.lax.dot_general(
        ctx_all, Wo_ref[...], (((1,), (0,)), ((), ())),
        preferred_element_type=jnp.float32)


def kernel(physics_x, structural_x, W_struct, b_struct, gamma, beta,
           W_phys, Wq, Wk, Wv, Wo, atom_to_nuc):
    nuc_col = atom_to_nuc.astype(jnp.int32).reshape(_N, 1)

    h, q, k, v, fq, gk = pl.pallas_call(
        _embed_qkv,
        grid=(_N // _BA,),
        in_specs=[
            pl.BlockSpec((_BA, 10), lambda i: (i, 0)),
            pl.BlockSpec((_BA, 4), lambda i: (i, 0)),
            pl.BlockSpec((4, _H), lambda i: (0, 0)),
            pl.BlockSpec((1, _H), lambda i: (0, 0)),
            pl.BlockSpec((1, _H), lambda i: (0, 0)),
            pl.BlockSpec((1, _H), lambda i: (0, 0)),
            pl.BlockSpec((10, _H), lambda i: (0, 0)),
            pl.BlockSpec((_H, _H), lambda i: (0, 0)),
            pl.BlockSpec((_H, _H), lambda i: (0, 0)),
            pl.BlockSpec((_H, _H), lambda i: (0, 0)),
        ],
        out_specs=[
            pl.BlockSpec((_BA, _H), lambda i: (i, 0)),
            pl.BlockSpec((_BA, _H), lambda i: (i, 0)),
            pl.BlockSpec((_BA, _H), lambda i: (i, 0)),
            pl.BlockSpec((_BA, _H), lambda i: (i, 0)),
            pl.BlockSpec((_BA, 1), lambda i: (i, 0)),
            pl.BlockSpec((_BA, 1), lambda i: (i, 0)),
        ],
        out_shape=[
            jax.ShapeDtypeStruct((_N, _H), jnp.float32),
            jax.ShapeDtypeStruct((_N, _H), jnp.bfloat16),
            jax.ShapeDtypeStruct((_N, _H), jnp.bfloat16),
            jax.ShapeDtypeStruct((_N, _H), jnp.float32),
            jax.ShapeDtypeStruct((_N, 1), jnp.int32),
            jax.ShapeDtypeStruct((_N, 1), jnp.int32),
        ],
    )(physics_x, structural_x, W_struct, b_struct.reshape(1, _H),
      gamma.reshape(1, _H), beta.reshape(1, _H), W_phys,
      Wq.astype(jnp.bfloat16), Wk.astype(jnp.bfloat16),
      Wv.astype(jnp.bfloat16))

    gk_row = gk.reshape(1, _N)
    nuc_row = nuc_col.reshape(1, _N)

    out = pl.pallas_call(
        _attn,
        grid=(_N // _BQ,),
        in_specs=[
            pl.BlockSpec((_BQ, 1), lambda i: (i, 0)),
            pl.BlockSpec((1, _N), lambda i: (0, 0)),
            pl.BlockSpec((_BQ, 1), lambda i: (i, 0)),
            pl.BlockSpec((1, _N), lambda i: (0, 0)),
            pl.BlockSpec((_BQ, _H), lambda i: (i, 0)),
            pl.BlockSpec((_N, _H), lambda i: (0, 0)),
            pl.BlockSpec((_N, _H), lambda i: (0, 0)),
            pl.BlockSpec((_BQ, _H), lambda i: (i, 0)),
            pl.BlockSpec((_H, _H), lambda i: (0, 0)),
        ],
        out_specs=pl.BlockSpec((_BQ, _H), lambda i: (i, 0)),
        out_shape=jax.ShapeDtypeStruct((_N, _H), jnp.float32),
    )(fq, gk_row, nuc_col, nuc_row, q, k, v, h, Wo.astype(jnp.bfloat16))
    return out


# speedup vs baseline: 1.1834x; 1.0627x over previous
import jax
import jax.numpy as jnp
from jax.experimental import pallas as pl
from jax.experimental.pallas import tpu as pltpu

_N, _H, _NH, _DH = 2048, 512, 8, 64
_BA = 256
_BQ = 256
_NP = _N // _BA
_NEG = -1e9
_SCALE = 0.125


def _body(px_ref, sx_ref, pxT_ref, sxT_ref, Ws_ref, bs_ref, g_ref, b_ref,
          Wp_ref, Wq_ref, Wk_ref, Wv_ref, Wo_ref, nc_ref, nr_ref,
          o_ref,
          hs, qs, ks, vs, fqs, gks):
    i = pl.program_id(0)

    @pl.when(i < _NP)
    def _embed():
        rows = pl.ds(i * _BA, _BA)
        px = px_ref[...]
        sx = sx_ref[...]
        h = jax.lax.dot_general(sx, Ws_ref[...], (((1,), (0,)), ((), ())),
                                preferred_element_type=jnp.float32)
        h = h + bs_ref[...]
        mu = jnp.mean(h, axis=1, keepdims=True)
        var = jnp.mean((h - mu) ** 2, axis=1, keepdims=True)
        h = (h - mu) / jnp.sqrt(var + 1e-5) * g_ref[...] + b_ref[...]
        h = h * jax.nn.sigmoid(h)
        h = h + jax.lax.dot_general(px, Wp_ref[...], (((1,), (0,)), ((), ())),
                                    preferred_element_type=jnp.float32)
        hs[rows, :] = h
        hb = h.astype(jnp.bfloat16)
        qs[rows, :] = (jax.lax.dot_general(
            hb, Wq_ref[...], (((1,), (0,)), ((), ())),
            preferred_element_type=jnp.float32) * _SCALE).astype(jnp.bfloat16)
        ks[rows, :] = jax.lax.dot_general(
            hb, Wk_ref[...], (((1,), (0,)), ((), ())),
            preferred_element_type=jnp.float32).astype(jnp.bfloat16)
        vs[rows, :] = jax.lax.dot_general(
            hb, Wv_ref[...], (((1,), (0,)), ((), ())),
            preferred_element_type=jnp.float32)
        d = (px[:, 6:7] > 0).astype(jnp.int32)
        a = (px[:, 7:8] > 0).astype(jnp.int32)
        ar = (sx[:, 1:2] > 0).astype(jnp.int32)
        fqs[rows, :] = d + 2 * a + 4 * ar
        cols = pl.ds(i * _BA, _BA)
        dr = (pxT_ref[6:7, :] > 0).astype(jnp.int32)
        ar_ = (pxT_ref[7:8, :] > 0).astype(jnp.int32)
        arr = (sxT_ref[1:2, :] > 0).astype(jnp.int32)
        gks[:, cols] = 2 * dr + ar_ + 4 * arr

    @pl.when(i >= _NP)
    def _attn():
        j = i - _NP
        rows = pl.ds(j * _BQ, _BQ)
        q = qs[rows, :]
        k = ks[...]
        v = vs[...]
        mask = ((fqs[rows, :] & gks[...]) != 0) & (nc_ref[...] != nr_ref[...])
        parts = []
        for hh in range(_NH):
            sl = slice(hh * _DH, (hh + 1) * _DH)
            s = jax.lax.dot_general(q[:, sl], k[:, sl],
                                    (((1,), (1,)), ((), ())),
                                    preferred_element_type=jnp.float32)
            s = jnp.where(mask, s, _NEG)
            m = jnp.max(s, axis=1, keepdims=True)
            p = jnp.exp(s - m)
            l = jnp.sum(p, axis=1, keepdims=True)
            ctx = jax.lax.dot_general(p, v[:, sl], (((1,), (0,)), ((), ())),
                                      preferred_element_type=jnp.float32)
            parts.append(ctx / l)
        ctx_all = jnp.concatenate(parts, axis=1).astype(jnp.bfloat16)
        o_ref[...] = hs[rows, :] + jax.lax.dot_general(
            ctx_all, Wo_ref[...], (((1,), (0,)), ((), ())),
            preferred_element_type=jnp.float32)


def kernel(physics_x, structural_x, W_struct, b_struct, gamma, beta,
           W_phys, Wq, Wk, Wv, Wo, atom_to_nuc):
    nuc_col = atom_to_nuc.astype(jnp.int32).reshape(_N, 1)
    nuc_row = atom_to_nuc.astype(jnp.int32).reshape(1, _N)

    def _lo(i):
        return jnp.minimum(i, _NP - 1)

    def _hi(i):
        return jnp.maximum(i - _NP, 0)

    out = pl.pallas_call(
        _body,
        grid=(2 * _NP,),
        in_specs=[
            pl.BlockSpec((_BA, 10), lambda i: (_lo(i), 0)),
            pl.BlockSpec((_BA, 4), lambda i: (_lo(i), 0)),
            pl.BlockSpec((10, _BA), lambda i: (0, _lo(i))),
            pl.BlockSpec((4, _BA), lambda i: (0, _lo(i))),
            pl.BlockSpec((4, _H), lambda i: (0, 0)),
            pl.BlockSpec((1, _H), lambda i: (0, 0)),
            pl.BlockSpec((1, _H), lambda i: (0, 0)),
            pl.BlockSpec((1, _H), lambda i: (0, 0)),
            pl.BlockSpec((10, _H), lambda i: (0, 0)),
            pl.BlockSpec((_H, _H), lambda i: (0, 0)),
            pl.BlockSpec((_H, _H), lambda i: (0, 0)),
            pl.BlockSpec((_H, _H), lambda i: (0, 0)),
            pl.BlockSpec((_H, _H), lambda i: (0, 0)),
            pl.BlockSpec((_BQ, 1), lambda i: (_hi(i), 0)),
            pl.BlockSpec((1, _N), lambda i: (0, 0)),
        ],
        out_specs=pl.BlockSpec((_BQ, _H), lambda i: (_hi(i), 0)),
        out_shape=jax.ShapeDtypeStruct((_N, _H), jnp.float32),
        scratch_shapes=[
            pltpu.VMEM((_N, _H), jnp.float32),
            pltpu.VMEM((_N, _H), jnp.bfloat16),
            pltpu.VMEM((_N, _H), jnp.bfloat16),
            pltpu.VMEM((_N, _H), jnp.float32),
            pltpu.VMEM((_N, 1), jnp.int32),
            pltpu.VMEM((1, _N), jnp.int32),
        ],
    )(physics_x, structural_x, physics_x.T, structural_x.T,
      W_struct, b_struct.reshape(1, _H), gamma.reshape(1, _H),
      beta.reshape(1, _H), W_phys, Wq.astype(jnp.bfloat16),
      Wk.astype(jnp.bfloat16), Wv.astype(jnp.bfloat16),
      Wo.astype(jnp.bfloat16), nuc_col, nuc_row)
    return out
